# baseline (speedup 1.0000x reference)
import functools

import jax
import jax.numpy as jnp
from jax import lax
from jax.experimental import pallas as pl
from jax.experimental.pallas import tpu as pltpu
from jax.experimental.pallas import tpu_sc as plsc

_TABLE_SIZE = 256
_X_RANGE = 3.0
_NUM_WORKERS = 32
_ROWS = 8
_CHUNK_COLS = 2048
_LANES = 16


def _tec_body(x_hbm, t_hbm, d_hbm, out_hbm, t_v, d_v, in_v, out_v,
              sem_in0, sem_in1, sem_out0, sem_out1,
              *, col_span, n_chunks):
    wid = lax.axis_index("s") * 2 + lax.axis_index("c")
    row0 = (wid // 4) * _ROWS
    col0 = (wid % 4) * col_span
    sems_in = (sem_in0, sem_in1)
    sems_out = (sem_out0, sem_out1)

    pltpu.sync_copy(t_hbm, t_v)
    pltpu.sync_copy(d_hbm, d_v)

    scale = jnp.float32((_TABLE_SIZE - 1) / (2.0 * _X_RANGE))
    shift = jnp.float32((_TABLE_SIZE - 1) / 2.0)

    def fetch(c, b):
        pltpu.async_copy(
            x_hbm.at[pl.ds(row0, _ROWS),
                     pl.ds(col0 + c * _CHUNK_COLS, _CHUNK_COLS)],
            in_v.at[b], sems_in[b])

    fetch(0, 0)
    fetch(1, 1)

    n_vecs = _ROWS * _CHUNK_COLS // _LANES
    vecs_per_row = _CHUNK_COLS // _LANES

    def pair_body(p, carry):
        for b in range(2):
            c = p * 2 + b
            pltpu.make_async_copy(
                x_hbm.at[pl.ds(0, _ROWS), pl.ds(0, _CHUNK_COLS)],
                in_v.at[b], sems_in[b]).wait()
            @pl.when(p > 0)
            def _():
                pltpu.make_async_copy(
                    out_v.at[b],
                    out_hbm.at[pl.ds(0, _ROWS), pl.ds(0, _CHUNK_COLS)],
                    sems_out[b]).wait()

            @plsc.parallel_loop(0, n_vecs, unroll=16)
            def _(i):
                r = i // vecs_per_row
                j = i % vecs_per_row
                xv = in_v[b, r, pl.ds(j * _LANES, _LANES)]
                idx = jnp.minimum(
                    jnp.maximum(xv * scale + shift, 0.0),
                    jnp.float32(_TABLE_SIZE - 1))
                i0 = idx.astype(jnp.int32)
                va = plsc.load_gather(t_v, [i0])
                vb = plsc.load_gather(d_v, [i0])
                out_v[b, r, pl.ds(j * _LANES, _LANES)] = va + idx * vb

            pltpu.async_copy(
                out_v.at[b],
                out_hbm.at[pl.ds(row0, _ROWS),
                           pl.ds(col0 + c * _CHUNK_COLS, _CHUNK_COLS)],
                sems_out[b])

            @pl.when(c + 2 < n_chunks)
            def _():
                fetch(c + 2, b)
        return carry

    lax.fori_loop(0, n_chunks // 2, pair_body, 0)

    for b in range(2):
        pltpu.make_async_copy(
            out_v.at[b], out_hbm.at[pl.ds(0, _ROWS), pl.ds(0, _CHUNK_COLS)],
            sems_out[b]).wait()


def kernel(x, table):
    n_rows, n_cols = x.shape
    assert n_rows % _ROWS == 0
    row_groups = n_rows // _ROWS
    col_splits = _NUM_WORKERS // row_groups
    col_span = n_cols // col_splits
    n_chunks = col_span // _CHUNK_COLS
    assert col_span * col_splits == n_cols
    assert n_chunks * _CHUNK_COLS == col_span and n_chunks % 2 == 0

    dtable = jnp.concatenate(
        [table[1:] - table[:-1], jnp.zeros((1,), jnp.float32)])
    atable = table - jnp.arange(_TABLE_SIZE, dtype=jnp.float32) * dtable

    mesh = plsc.VectorSubcoreMesh(core_axis_name="c", subcore_axis_name="s")
    body = functools.partial(_tec_body, col_span=col_span, n_chunks=n_chunks)
    out = pl.kernel(
        body,
        mesh=mesh,
        compiler_params=pltpu.CompilerParams(needs_layout_passes=False),
        out_type=jax.ShapeDtypeStruct((n_rows, n_cols), jnp.float32),
        scratch_types=[
            pltpu.VMEM((_TABLE_SIZE,), jnp.float32),
            pltpu.VMEM((_TABLE_SIZE,), jnp.float32),
            pltpu.VMEM((2, _ROWS, _CHUNK_COLS), jnp.float32),
            pltpu.VMEM((2, _ROWS, _CHUNK_COLS), jnp.float32),
            pltpu.SemaphoreType.DMA,
            pltpu.SemaphoreType.DMA,
            pltpu.SemaphoreType.DMA,
            pltpu.SemaphoreType.DMA,
        ],
    )(x, atable, dtable)
    return out


# speedup vs baseline: 1.2101x; 1.2101x over previous
import functools

import jax
import jax.numpy as jnp
from jax import lax
from jax.experimental import pallas as pl
from jax.experimental.pallas import tpu as pltpu
from jax.experimental.pallas import tpu_sc as plsc

_TABLE_SIZE = 256
_X_RANGE = 3.0
_NUM_WORKERS = 32
_ROWS = 8
_CHUNK_COLS = 2048
_LANES = 16


def _tec_body(x_hbm, t_hbm, d_hbm, out_hbm, t_v, d_v, in_v, out_v,
              sem_in0, sem_in1, sem_out0, sem_out1,
              *, col_span, n_chunks):
    wid = lax.axis_index("s") * 2 + lax.axis_index("c")
    row0 = (wid // 4) * _ROWS
    col0 = (wid % 4) * col_span
    sems_in = (sem_in0, sem_in1)
    sems_out = (sem_out0, sem_out1)

    scale = jnp.float32((_TABLE_SIZE - 1) / (2.0 * _X_RANGE))
    shift = jnp.float32((_TABLE_SIZE - 1) / 2.0)

    def fetch(c, b):
        pltpu.async_copy(
            x_hbm.at[pl.ds(row0, _ROWS),
                     pl.ds(col0 + c * _CHUNK_COLS, _CHUNK_COLS)],
            in_v.at[b], sems_in[b])

    fetch(0, 0)
    fetch(1, 1)
    tab_copy = pltpu.make_async_copy(t_hbm, t_v, sems_out[0])
    tab_copy.start()
    dtab_copy = pltpu.make_async_copy(d_hbm, d_v, sems_out[1])
    dtab_copy.start()
    tab_copy.wait()
    dtab_copy.wait()

    n_vecs = _ROWS * _CHUNK_COLS // _LANES
    vecs_per_row = _CHUNK_COLS // _LANES

    def pair_body(p, carry):
        for b in range(2):
            c = p * 2 + b
            pltpu.make_async_copy(
                x_hbm.at[pl.ds(0, _ROWS), pl.ds(0, _CHUNK_COLS)],
                in_v.at[b], sems_in[b]).wait()
            @pl.when(p > 0)
            def _():
                pltpu.make_async_copy(
                    out_v.at[b],
                    out_hbm.at[pl.ds(0, _ROWS), pl.ds(0, _CHUNK_COLS)],
                    sems_out[b]).wait()

            @plsc.parallel_loop(0, n_vecs, unroll=8)
            def _(i):
                r = i // vecs_per_row
                j = i % vecs_per_row
                xv = in_v[b, r, pl.ds(j * _LANES, _LANES)]
                idx = jnp.minimum(
                    jnp.maximum(xv * scale + shift, 0.0),
                    jnp.float32(_TABLE_SIZE - 1))
                i0 = idx.astype(jnp.int32)
                va = plsc.load_gather(t_v, [i0])
                vb = plsc.load_gather(d_v, [i0])
                out_v[b, r, pl.ds(j * _LANES, _LANES)] = va + idx * vb

            pltpu.async_copy(
                out_v.at[b],
                out_hbm.at[pl.ds(row0, _ROWS),
                           pl.ds(col0 + c * _CHUNK_COLS, _CHUNK_COLS)],
                sems_out[b])

            @pl.when(c + 2 < n_chunks)
            def _():
                fetch(c + 2, b)
        return carry

    lax.fori_loop(0, n_chunks // 2, pair_body, 0)

    for b in range(2):
        pltpu.make_async_copy(
            out_v.at[b], out_hbm.at[pl.ds(0, _ROWS), pl.ds(0, _CHUNK_COLS)],
            sems_out[b]).wait()


def kernel(x, table):
    n_rows, n_cols = x.shape
    assert n_rows % _ROWS == 0
    row_groups = n_rows // _ROWS
    col_splits = _NUM_WORKERS // row_groups
    col_span = n_cols // col_splits
    n_chunks = col_span // _CHUNK_COLS
    assert col_span * col_splits == n_cols
    assert n_chunks * _CHUNK_COLS == col_span and n_chunks % 2 == 0

    dtable = jnp.concatenate(
        [table[1:] - table[:-1], jnp.zeros((1,), jnp.float32)])
    atable = table - jnp.arange(_TABLE_SIZE, dtype=jnp.float32) * dtable

    mesh = plsc.VectorSubcoreMesh(core_axis_name="c", subcore_axis_name="s")
    body = functools.partial(_tec_body, col_span=col_span, n_chunks=n_chunks)
    out = pl.kernel(
        body,
        mesh=mesh,
        compiler_params=pltpu.CompilerParams(needs_layout_passes=False),
        out_type=jax.ShapeDtypeStruct((n_rows, n_cols), jnp.float32),
        scratch_types=[
            pltpu.VMEM((_TABLE_SIZE,), jnp.float32),
            pltpu.VMEM((_TABLE_SIZE,), jnp.float32),
            pltpu.VMEM((2, _ROWS, _CHUNK_COLS), jnp.float32),
            pltpu.VMEM((2, _ROWS, _CHUNK_COLS), jnp.float32),
            pltpu.SemaphoreType.DMA,
            pltpu.SemaphoreType.DMA,
            pltpu.SemaphoreType.DMA,
            pltpu.SemaphoreType.DMA,
        ],
    )(x, atable, dtable)
    return out


# speedup vs baseline: 1.2103x; 1.0002x over previous
import functools

import jax
import jax.numpy as jnp
from jax import lax
from jax.experimental import pallas as pl
from jax.experimental.pallas import tpu as pltpu
from jax.experimental.pallas import tpu_sc as plsc

_TABLE_SIZE = 256
_X_RANGE = 3.0
_NUM_WORKERS = 32
_ROWS = 8
_CHUNK_COLS = 2048
_LANES = 16


def _tec_body(x_hbm, t_hbm, d_hbm, out_hbm, t_v, d_v, in_v, out_v,
              sem_in0, sem_in1, sem_out0, sem_out1,
              *, col_span, n_chunks):
    wid = lax.axis_index("s") * 2 + lax.axis_index("c")
    row0 = (wid // 4) * _ROWS
    col0 = (wid % 4) * col_span
    sems_in = (sem_in0, sem_in1)
    sems_out = (sem_out0, sem_out1)

    scale = jnp.float32((_TABLE_SIZE - 1) / (2.0 * _X_RANGE))
    shift = jnp.float32((_TABLE_SIZE - 1) / 2.0)

    def fetch(c, b):
        pltpu.async_copy(
            x_hbm.at[pl.ds(row0, _ROWS),
                     pl.ds(col0 + c * _CHUNK_COLS, _CHUNK_COLS)],
            in_v.at[b], sems_in[b])

    fetch(0, 0)
    fetch(1, 1)
    tab_copy = pltpu.make_async_copy(t_hbm, t_v, sems_out[0])
    tab_copy.start()
    dtab_copy = pltpu.make_async_copy(d_hbm, d_v, sems_out[1])
    dtab_copy.start()
    tab_copy.wait()
    dtab_copy.wait()

    n_vecs = _ROWS * _CHUNK_COLS // _LANES
    vecs_per_row = _CHUNK_COLS // _LANES

    def pair_body(p, carry):
        for b in range(2):
            c = p * 2 + b
            pltpu.make_async_copy(
                x_hbm.at[pl.ds(0, _ROWS), pl.ds(0, _CHUNK_COLS)],
                in_v.at[b], sems_in[b]).wait()
            @pl.when(p > 0)
            def _():
                pltpu.make_async_copy(
                    out_v.at[b],
                    out_hbm.at[pl.ds(0, _ROWS), pl.ds(0, _CHUNK_COLS)],
                    sems_out[b]).wait()

            @plsc.parallel_loop(0, n_vecs, unroll=8)
            def _(i):
                r = i // vecs_per_row
                j = i % vecs_per_row
                xv = in_v[b, r, pl.ds(j * _LANES, _LANES)]
                idx = jnp.minimum(
                    jnp.maximum(xv * scale + shift, 0.0),
                    jnp.float32(_TABLE_SIZE - 1))
                i0 = idx.astype(jnp.int32)
                va = plsc.load_gather(t_v, [i0])
                vb = plsc.load_gather(d_v, [i0])
                out_v[b, r, pl.ds(j * _LANES, _LANES)] = va + idx * vb

            pltpu.async_copy(
                out_v.at[b],
                out_hbm.at[pl.ds(row0, _ROWS),
                           pl.ds(col0 + c * _CHUNK_COLS, _CHUNK_COLS)],
                sems_out[b])

            @pl.when(c + 2 < n_chunks)
            def _():
                fetch(c + 2, b)
        return carry

    lax.fori_loop(0, n_chunks // 2, pair_body, 0)

    for b in range(2):
        pltpu.make_async_copy(
            out_v.at[b], out_hbm.at[pl.ds(0, _ROWS), pl.ds(0, _CHUNK_COLS)],
            sems_out[b]).wait()


def kernel(x, table):
    n_rows, n_cols = x.shape
    assert n_rows % _ROWS == 0
    row_groups = n_rows // _ROWS
    col_splits = _NUM_WORKERS // row_groups
    col_span = n_cols // col_splits
    n_chunks = col_span // _CHUNK_COLS
    assert col_span * col_splits == n_cols
    assert n_chunks * _CHUNK_COLS == col_span and n_chunks % 2 == 0

    dtable = jnp.concatenate(
        [table[1:] - table[:-1], jnp.zeros((1,), jnp.float32)])
    atable = table - jnp.arange(_TABLE_SIZE, dtype=jnp.float32) * dtable

    mesh = plsc.VectorSubcoreMesh(core_axis_name="c", subcore_axis_name="s")
    body = functools.partial(_tec_body, col_span=col_span, n_chunks=n_chunks)
    out = pl.kernel(
        body,
        mesh=mesh,
        compiler_params=pltpu.CompilerParams(needs_layout_passes=False,
                                             skip_device_barrier=True),
        out_type=jax.ShapeDtypeStruct((n_rows, n_cols), jnp.float32),
        scratch_types=[
            pltpu.VMEM((_TABLE_SIZE,), jnp.float32),
            pltpu.VMEM((_TABLE_SIZE,), jnp.float32),
            pltpu.VMEM((2, _ROWS, _CHUNK_COLS), jnp.float32),
            pltpu.VMEM((2, _ROWS, _CHUNK_COLS), jnp.float32),
            pltpu.SemaphoreType.DMA,
            pltpu.SemaphoreType.DMA,
            pltpu.SemaphoreType.DMA,
            pltpu.SemaphoreType.DMA,
        ],
    )(x, atable, dtable)
    return out
